# baseline (speedup 1.0000x reference)
import functools

import jax
import jax.numpy as jnp
from jax import lax
from jax.experimental import pallas as pl
from jax.experimental.pallas import tpu as pltpu
from jax.experimental.pallas import tpu_sc as plsc

_LANES = 128
_TOK_BLK = 512


@functools.lru_cache(maxsize=None)
def _flags_gather(rows, nc, ns):
    nw = nc * ns
    rows_w = rows // nw
    mesh = plsc.VectorSubcoreMesh(core_axis_name="c", subcore_axis_name="s")

    def body(ids_hbm, table_hbm, out_hbm, idx_v, fl_v, sem):
        wid = lax.axis_index("s") * nc + lax.axis_index("c")
        r0 = wid * rows_w
        pltpu.sync_copy(ids_hbm.at[pl.ds(r0, rows_w)], idx_v)
        copies = [
            pltpu.async_copy(table_hbm.at[idx_v.at[j]], fl_v.at[j], sem)
            for j in range(rows_w)
        ]
        for cp in copies:
            cp.wait()
        pltpu.sync_copy(fl_v, out_hbm.at[pl.ds(r0, rows_w)])

    return pl.kernel(
        body,
        out_type=jax.ShapeDtypeStruct((rows, _LANES), jnp.int32),
        mesh=mesh,
        scratch_types=[
            pltpu.VMEM((rows_w, _LANES), jnp.int32),
            pltpu.VMEM((rows_w, _LANES), jnp.int32),
            pltpu.SemaphoreType.DMA,
        ],
    )


def _add_body(fl_ref, se_ref, emb_ref, out_ref):
    f = fl_ref[...]
    e0 = se_ref[0:1, :]
    e1 = se_ref[1:2, :]
    e2 = se_ref[2:3, :]
    sp = jnp.where(f == 1, e1, jnp.where(f == 2, e2, e0))
    out_ref[...] = emb_ref[...] + sp


def kernel(token_embeds, token_ids, special_flags, special_emb):
    b, t, d = token_embeds.shape
    n = b * t
    info = plsc.get_sparse_core_info()
    nc, ns = info.num_cores, info.num_subcores

    ids2d = token_ids.astype(jnp.int32).reshape(n // _LANES, _LANES)
    flags2d = _flags_gather(n // _LANES, nc, ns)(
        ids2d, special_flags.astype(jnp.int32))
    flags_col = flags2d.reshape(n, 1)

    emb2d = token_embeds.reshape(n, d)
    out2d = pl.pallas_call(
        _add_body,
        grid=(n // _TOK_BLK,),
        in_specs=[
            pl.BlockSpec((_TOK_BLK, 1), lambda i: (i, 0)),
            pl.BlockSpec((3, d), lambda i: (0, 0)),
            pl.BlockSpec((_TOK_BLK, d), lambda i: (i, 0)),
        ],
        out_specs=pl.BlockSpec((_TOK_BLK, d), lambda i: (i, 0)),
        out_shape=jax.ShapeDtypeStruct((n, d), jnp.float32),
        compiler_params=pltpu.CompilerParams(
            dimension_semantics=("arbitrary",)),
    )(flags_col, special_emb, emb2d)
    return out2d.reshape(b, t, d)


# speedup vs baseline: 1.2193x; 1.2193x over previous
import functools

import jax
import jax.numpy as jnp
from jax import lax
from jax.experimental import pallas as pl
from jax.experimental.pallas import tpu as pltpu
from jax.experimental.pallas import tpu_sc as plsc

_LANES = 128
_TOK_BLK = 2048


@functools.lru_cache(maxsize=None)
def _flags_gather(n, nc, ns):
    nw = nc * ns
    n_w = n // nw
    rows_w = n_w // _LANES
    mesh = plsc.VectorSubcoreMesh(core_axis_name="c", subcore_axis_name="s")

    def body(ids_hbm, table_hbm, out_hbm, idx_v, fl_v, sem):
        wid = lax.axis_index("s") * nc + lax.axis_index("c")
        r0 = wid * rows_w
        pltpu.sync_copy(ids_hbm.at[pl.ds(r0, rows_w)], idx_v)
        copies = [
            pltpu.async_copy(table_hbm.at[idx_v.at[j]],
                             fl_v.at[pl.ds(j * _LANES, _LANES)], sem)
            for j in range(rows_w)
        ]
        for cp in copies:
            cp.wait()
        pltpu.sync_copy(fl_v, out_hbm.at[0, pl.ds(wid * n_w, n_w)])

    return pl.kernel(
        body,
        out_type=jax.ShapeDtypeStruct((1, n), jnp.int32),
        mesh=mesh,
        scratch_types=[
            pltpu.VMEM((rows_w, _LANES), jnp.int32),
            pltpu.VMEM((n_w,), jnp.int32),
            pltpu.SemaphoreType.DMA,
        ],
    )


def _add_body(fl_ref, se_ref, emb_ref, out_ref):
    f = fl_ref[...]
    k8 = lax.broadcasted_iota(jnp.int32, (8, 1), 0)
    oh = (f == k8).astype(jnp.float32)
    sp = lax.dot_general(
        oh, se_ref[...],
        dimension_numbers=(((0,), (0,)), ((), ())),
        preferred_element_type=jnp.float32)
    out_ref[...] = emb_ref[...] + sp


def kernel(token_embeds, token_ids, special_flags, special_emb):
    b, t, d = token_embeds.shape
    n = b * t
    info = plsc.get_sparse_core_info()
    nc, ns = info.num_cores, info.num_subcores

    ids2d = token_ids.astype(jnp.int32).reshape(n // _LANES, _LANES)
    flags_row = _flags_gather(n, nc, ns)(
        ids2d, special_flags.astype(jnp.int32))

    se8 = jnp.zeros((8, d), jnp.float32).at[:3].set(special_emb)
    emb2d = token_embeds.reshape(n, d)
    out2d = pl.pallas_call(
        _add_body,
        grid=(n // _TOK_BLK,),
        in_specs=[
            pl.BlockSpec((1, _TOK_BLK), lambda i: (0, i)),
            pl.BlockSpec((8, d), lambda i: (0, 0)),
            pl.BlockSpec((_TOK_BLK, d), lambda i: (i, 0)),
        ],
        out_specs=pl.BlockSpec((_TOK_BLK, d), lambda i: (i, 0)),
        out_shape=jax.ShapeDtypeStruct((n, d), jnp.float32),
        compiler_params=pltpu.CompilerParams(
            dimension_semantics=("arbitrary",)),
    )(flags_row, se8, emb2d)
    return out2d.reshape(b, t, d)


# speedup vs baseline: 1.2324x; 1.0107x over previous
import functools

import jax
import jax.numpy as jnp
from jax import lax
from jax.experimental import pallas as pl
from jax.experimental.pallas import tpu as pltpu
from jax.experimental.pallas import tpu_sc as plsc

_LANES = 128
_TOK_BLK = 2048


@functools.lru_cache(maxsize=None)
def _flags_gather(n, nc, ns):
    nw = nc * ns
    n_w = n // nw
    rows_w = n_w // _LANES
    mesh = plsc.VectorSubcoreMesh(core_axis_name="c", subcore_axis_name="s")

    def body(ids_hbm, table_hbm, out_hbm, idx_v, fl_v, sem):
        wid = lax.axis_index("s") * nc + lax.axis_index("c")
        r0 = wid * rows_w
        pltpu.sync_copy(ids_hbm.at[pl.ds(r0, rows_w)], idx_v)
        copies = [
            pltpu.async_copy(table_hbm.at[idx_v.at[j]],
                             fl_v.at[pl.ds(j * _LANES, _LANES)], sem)
            for j in range(rows_w)
        ]
        for cp in copies:
            cp.wait()
        pltpu.sync_copy(fl_v, out_hbm.at[0, pl.ds(wid * n_w, n_w)])

    return pl.kernel(
        body,
        out_type=jax.ShapeDtypeStruct((1, n), jnp.int32),
        mesh=mesh,
        scratch_types=[
            pltpu.VMEM((rows_w, _LANES), jnp.int32),
            pltpu.VMEM((n_w,), jnp.int32),
            pltpu.SemaphoreType.DMA,
        ],
    )


def _add_body(fl_ref, se_ref, emb_ref, out_ref):
    f = fl_ref[...]
    any_special = jnp.sum(f) > 0

    @pl.when(any_special)
    def _dense():
        k8 = lax.broadcasted_iota(jnp.int32, (8, 1), 0)
        oh = (f == k8).astype(jnp.float32)
        sp = lax.dot_general(
            oh, se_ref[...],
            dimension_numbers=(((0,), (0,)), ((), ())),
            preferred_element_type=jnp.float32)
        out_ref[...] = emb_ref[...] + sp

    @pl.when(jnp.logical_not(any_special))
    def _copy():
        out_ref[...] = emb_ref[...]


def kernel(token_embeds, token_ids, special_flags, special_emb):
    b, t, d = token_embeds.shape
    n = b * t
    info = plsc.get_sparse_core_info()
    nc, ns = info.num_cores, info.num_subcores

    ids2d = token_ids.astype(jnp.int32).reshape(n // _LANES, _LANES)
    flags_row = _flags_gather(n, nc, ns)(
        ids2d, special_flags.astype(jnp.int32))

    se8 = jnp.zeros((8, d), jnp.float32).at[:3].set(special_emb)
    emb2d = token_embeds.reshape(n, d)
    out2d = pl.pallas_call(
        _add_body,
        grid=(n // _TOK_BLK,),
        in_specs=[
            pl.BlockSpec((1, _TOK_BLK), lambda i: (0, i)),
            pl.BlockSpec((8, d), lambda i: (0, 0)),
            pl.BlockSpec((_TOK_BLK, d), lambda i: (i, 0)),
        ],
        out_specs=pl.BlockSpec((_TOK_BLK, d), lambda i: (i, 0)),
        out_shape=jax.ShapeDtypeStruct((n, d), jnp.float32),
        compiler_params=pltpu.CompilerParams(
            dimension_semantics=("arbitrary",)),
    )(flags_row, se8, emb2d)
    return out2d.reshape(b, t, d)


# speedup vs baseline: 1.2337x; 1.0011x over previous
import functools

import jax
import jax.numpy as jnp
from jax import lax
from jax.experimental import pallas as pl
from jax.experimental.pallas import tpu as pltpu
from jax.experimental.pallas import tpu_sc as plsc

_LANES = 128
_TOK_BLK = 2048


@functools.lru_cache(maxsize=None)
def _flags_gather(b, t, nc, ns):
    n = b * t
    nw = nc * ns
    n_w = n // nw
    rows_w = n_w // _LANES
    w_per_b = t // n_w
    mesh = plsc.VectorSubcoreMesh(core_axis_name="c", subcore_axis_name="s")

    def body(ids_hbm, table_hbm, out_hbm, idx_v, fl_v, sem):
        wid = lax.axis_index("s") * nc + lax.axis_index("c")
        bi = wid // w_per_b
        toff = (wid % w_per_b) * n_w
        pltpu.sync_copy(ids_hbm.at[bi, pl.ds(toff, n_w)], idx_v)
        copies = [
            pltpu.async_copy(
                table_hbm.at[idx_v.at[pl.ds(j * _LANES, _LANES)]],
                fl_v.at[pl.ds(j * _LANES, _LANES)], sem)
            for j in range(rows_w)
        ]
        for cp in copies:
            cp.wait()
        pltpu.sync_copy(fl_v, out_hbm.at[0, pl.ds(wid * n_w, n_w)])

    return pl.kernel(
        body,
        out_type=jax.ShapeDtypeStruct((1, n), jnp.int32),
        mesh=mesh,
        scratch_types=[
            pltpu.VMEM((n_w,), jnp.int32),
            pltpu.VMEM((n_w,), jnp.int32),
            pltpu.SemaphoreType.DMA,
        ],
    )


def _add_body(fl_ref, se_ref, emb_ref, out_ref):
    f = fl_ref[...]
    any_special = jnp.sum(f) > 0

    @pl.when(any_special)
    def _dense():
        k8 = lax.broadcasted_iota(jnp.int32, (8, 1), 0)
        oh = (f == k8).astype(jnp.float32)
        sp = lax.dot_general(
            oh, se_ref[...],
            dimension_numbers=(((0,), (0,)), ((), ())),
            preferred_element_type=jnp.float32)
        out_ref[...] = emb_ref[...] + sp

    @pl.when(jnp.logical_not(any_special))
    def _copy():
        out_ref[...] = emb_ref[...]


def kernel(token_embeds, token_ids, special_flags, special_emb):
    b, t, d = token_embeds.shape
    n = b * t
    info = plsc.get_sparse_core_info()
    nc, ns = info.num_cores, info.num_subcores

    flags_row = _flags_gather(b, t, nc, ns)(
        token_ids.astype(jnp.int32), special_flags.astype(jnp.int32))

    se8 = jnp.zeros((8, d), jnp.float32).at[:3].set(special_emb)
    emb2d = token_embeds.reshape(n, d)
    out2d = pl.pallas_call(
        _add_body,
        grid=(n // _TOK_BLK,),
        in_specs=[
            pl.BlockSpec((1, _TOK_BLK), lambda i: (0, i)),
            pl.BlockSpec((8, d), lambda i: (0, 0)),
            pl.BlockSpec((_TOK_BLK, d), lambda i: (i, 0)),
        ],
        out_specs=pl.BlockSpec((_TOK_BLK, d), lambda i: (i, 0)),
        out_shape=jax.ShapeDtypeStruct((n, d), jnp.float32),
        compiler_params=pltpu.CompilerParams(
            dimension_semantics=("arbitrary",)),
    )(flags_row, se8, emb2d)
    return out2d.reshape(b, t, d)
